# Initial kernel scaffold; baseline (speedup 1.0000x reference)
import jax
import jax.numpy as jnp
from jax.experimental import pallas as pl


def kernel(x_nchw, dlatents, w0, w0_mul, b0, w1, w1_mul, b1, b_mul, nw1, nw2, noise1, noise2, s1_w, s1_b, s2_w, s2_b, s_mul):
    raise NotImplementedError("write your pallas kernel here")


# speedup vs baseline: 1.0360x; 1.0360x over previous
import jax
import jax.numpy as jnp
from jax.experimental import pallas as pl
from jax.experimental.pallas import tpu as pltpu

_NEG_SLOPE = 0.2
_EPS = 1e-5


def _make_fused_kernel(H, W, C):
    H2, W2 = 2 * H, 2 * W

    def _conv9(src_ref, w_ref):
        acc = jnp.zeros((H2 * W2, C), jnp.float32)
        for i in range(3):
            for j in range(3):
                tap = src_ref[i:i + H2, j:j + W2, :].reshape(H2 * W2, C)
                acc = acc + jnp.dot(tap, w_ref[i, j],
                                    preferred_element_type=jnp.float32)
        return acc.reshape(H2, W2, C)

    def _epilogue(y, nz_ref, nw_ref, sc_ref, sh_ref):
        y = y + nw_ref[...].reshape(1, 1, C) * nz_ref[0]
        y = jnp.where(y >= 0.0, y, _NEG_SLOPE * y)
        mean = jnp.mean(y, axis=(0, 1), keepdims=True)
        var = jnp.mean(jnp.square(y - mean), axis=(0, 1), keepdims=True)
        y = (y - mean) * jax.lax.rsqrt(var + _EPS)
        return y * sc_ref[...].reshape(1, 1, C) + sh_ref[...].reshape(1, 1, C)

    def _body(x_ref, w0_ref, b0_ref, nz1_ref, nw1_ref, sc1_ref, sh1_ref,
              w1_ref, b1_ref, nz2_ref, nw2_ref, sc2_ref, sh2_ref,
              o_ref, up_ref, p_ref):
        up_ref[0:1] = jnp.zeros((1, W2 + 2, C), jnp.bfloat16)
        up_ref[H2 + 1:H2 + 2] = jnp.zeros((1, W2 + 2, C), jnp.bfloat16)
        up_ref[:, 0:1] = jnp.zeros((H2 + 2, 1, C), jnp.bfloat16)
        up_ref[:, W2 + 1:W2 + 2] = jnp.zeros((H2 + 2, 1, C), jnp.bfloat16)
        p_ref[0:1] = jnp.zeros((1, W2 + 2, C), jnp.float32)
        p_ref[H2 + 1:H2 + 2] = jnp.zeros((1, W2 + 2, C), jnp.float32)
        p_ref[:, 0:1] = jnp.zeros((H2 + 2, 1, C), jnp.float32)
        p_ref[:, W2 + 1:W2 + 2] = jnp.zeros((H2 + 2, 1, C), jnp.float32)

        x = x_ref[0].astype(jnp.bfloat16)
        up = jnp.repeat(jnp.repeat(x, 2, axis=0), 2, axis=1)
        up_ref[1:H2 + 1, 1:W2 + 1, :] = up

        y = _conv9(up_ref, w0_ref)
        p_ref[1:H2 + 1, 1:W2 + 1, :] = y
        v = 0.5 * y + 0.25 * (p_ref[0:H2, 1:W2 + 1, :] + p_ref[2:H2 + 2, 1:W2 + 1, :])
        p_ref[1:H2 + 1, 1:W2 + 1, :] = v
        y = 0.5 * v + 0.25 * (p_ref[1:H2 + 1, 0:W2, :] + p_ref[1:H2 + 1, 2:W2 + 2, :])
        y = y + b0_ref[...].reshape(1, 1, C)

        y = _epilogue(y, nz1_ref, nw1_ref, sc1_ref, sh1_ref)

        up_ref[1:H2 + 1, 1:W2 + 1, :] = y.astype(jnp.bfloat16)
        y = _conv9(up_ref, w1_ref) + b1_ref[...].reshape(1, 1, C)

        y = _epilogue(y, nz2_ref, nw2_ref, sc2_ref, sh2_ref)
        o_ref[0] = y

    return _body


def _style_affine(latent, w, b, w_mul, C):
    style = jnp.matmul(latent, (w * w_mul).T,
                       precision=jax.lax.Precision.HIGHEST) + b
    return style[:, :C] + 1.0, style[:, C:]


@jax.jit
def _forward(x_nchw, dlatents, params):
    N, Ci, H, W = x_nchw.shape
    Co = params["w0"].shape[0]
    H2, W2 = 2 * H, 2 * W

    x = jnp.transpose(x_nchw, (0, 2, 3, 1))

    w0 = (jnp.transpose(params["w0"], (2, 3, 1, 0)) * params["w0_mul"]).astype(jnp.bfloat16)
    w1 = (jnp.transpose(params["w1"], (2, 3, 1, 0)) * params["w1_mul"]).astype(jnp.bfloat16)
    b0 = params["b0"].reshape(1, Co) * params["b_mul"]
    b1 = params["b1"].reshape(1, Co) * params["b_mul"]
    sc1, sh1 = _style_affine(dlatents[:, 0], params["s1_w"], params["s1_b"],
                             params["s_mul"], Co)
    sc2, sh2 = _style_affine(dlatents[:, 1], params["s2_w"], params["s2_b"],
                             params["s_mul"], Co)
    nz1 = params["noise1"].reshape(N, H2, W2, 1)
    nz2 = params["noise2"].reshape(N, H2, W2, 1)

    y = pl.pallas_call(
        _make_fused_kernel(H, W, Co),
        out_shape=jax.ShapeDtypeStruct((N, H2, W2, Co), x.dtype),
        grid_spec=pltpu.PrefetchScalarGridSpec(
            num_scalar_prefetch=0,
            grid=(N,),
            in_specs=[
                pl.BlockSpec((1, H, W, Ci), lambda n: (n, 0, 0, 0)),
                pl.BlockSpec((3, 3, Ci, Co), lambda n: (0, 0, 0, 0)),
                pl.BlockSpec((1, Co), lambda n: (0, 0)),
                pl.BlockSpec((1, H2, W2, 1), lambda n: (n, 0, 0, 0)),
                pl.BlockSpec((1, 1, Co), lambda n: (0, 0, 0)),
                pl.BlockSpec((1, 1, Co), lambda n: (n, 0, 0)),
                pl.BlockSpec((1, 1, Co), lambda n: (n, 0, 0)),
                pl.BlockSpec((3, 3, Co, Co), lambda n: (0, 0, 0, 0)),
                pl.BlockSpec((1, Co), lambda n: (0, 0)),
                pl.BlockSpec((1, H2, W2, 1), lambda n: (n, 0, 0, 0)),
                pl.BlockSpec((1, 1, Co), lambda n: (0, 0, 0)),
                pl.BlockSpec((1, 1, Co), lambda n: (n, 0, 0)),
                pl.BlockSpec((1, 1, Co), lambda n: (n, 0, 0)),
            ],
            out_specs=pl.BlockSpec((1, H2, W2, Co), lambda n: (n, 0, 0, 0)),
            scratch_shapes=[
                pltpu.VMEM((H2 + 2, W2 + 2, Ci), jnp.bfloat16),
                pltpu.VMEM((H2 + 2, W2 + 2, Co), jnp.float32),
            ],
        ),
        compiler_params=pltpu.CompilerParams(dimension_semantics=("parallel",)),
    )(x, w0, b0, nz1, params["nw1"].reshape(1, 1, Co),
      sc1.reshape(N, 1, Co), sh1.reshape(N, 1, Co),
      w1, b1, nz2, params["nw2"].reshape(1, 1, Co),
      sc2.reshape(N, 1, Co), sh2.reshape(N, 1, Co))

    return jnp.transpose(y, (0, 3, 1, 2))


def kernel(x_nchw, dlatents, w0, w0_mul, b0, w1, w1_mul, b1, b_mul,
           nw1, nw2, noise1, noise2, s1_w, s1_b, s2_w, s2_b, s_mul):
    params = {"w0": w0, "w0_mul": w0_mul, "b0": b0, "w1": w1, "w1_mul": w1_mul,
              "b1": b1, "b_mul": b_mul, "nw1": nw1, "nw2": nw2,
              "noise1": noise1, "noise2": noise2, "s1_w": s1_w, "s1_b": s1_b,
              "s2_w": s2_w, "s2_b": s2_b, "s_mul": s_mul}
    return _forward(x_nchw, dlatents, params)
